# baseline (speedup 1.0000x reference)
import functools

import jax
import jax.numpy as jnp
import numpy as np
from jax import lax
from jax.experimental import pallas as pl
from jax.experimental.pallas import tpu as pltpu
from jax.experimental.pallas import tpu_sc as plsc

_INFO = plsc.get_sparse_core_info()
_NC = _INFO.num_cores
_NS = _INFO.num_subcores
_L = _INFO.num_lanes
_NW = _NC * _NS

_K = 128
_NB = 4
_TV = 128


def _wid():
    return lax.axis_index("s") * _NC + lax.axis_index("c")


def _diag_vecs():
    biota = lax.iota(jnp.int32, _L)
    cvecs = [(dg + biota) % _L for dg in range(_L)]
    return biota, cvecs


def _tr_body(v, d, tt_hbm, tail_hbm, mid_hbm, tin0, tin1, tout0, tout1,
             isem0, isem1, osem0, osem1):
    wid = _wid()
    tins, touts = (tin0, tin1), (tout0, tout1)
    isems, osems = (isem0, isem1), (osem0, osem1)
    nfull = v // _TV
    biota, cvecs = _diag_vecs()
    rshs = [lax.shift_right_logical(c, 1) for c in cvecs]
    cshs = [lax.shift_left(c & 1, 6) + biota for c in cvecs]

    def in_copies(t, b, fn):
        for fb in range(d // 8):
            getattr(pltpu.make_async_copy(
                tt_hbm.at[pl.ds(fb * 8, 8), pl.ds(t * _TV, _TV)],
                tins[b].at[pl.ds(fb * 8, 8)], isems[b]), fn)()

    def out_copy(t, b, fn):
        getattr(pltpu.make_async_copy(
            touts[b], mid_hbm.at[pl.ds(t * (_TV // 2), _TV // 2)],
            osems[b]), fn)()

    def transpose(b):
        src, dst = tins[b], touts[b]

        @plsc.parallel_loop(0, (d // _L) * (_TV // _L))
        def blk(u):
            rb = u // (_TV // _L)
            qb = u % (_TV // _L)
            rvec = biota + rb * _L
            for dg in range(_L):
                vals = plsc.load_gather(src, [rvec, cvecs[dg] + qb * _L])
                plsc.store_scatter(
                    dst, [rshs[dg] + qb * 8, cshs[dg] + rb * _L], vals)

    for i in range(2):
        in_copies(i * _NW + wid, i, "start")

    def it_body(g, carry):
        for b in range(2):
            i = g * 2 + b
            t = i * _NW + wid

            @pl.when((i >= 2) & (t - 2 * _NW < nfull))
            def _():
                out_copy(t - 2 * _NW, b, "wait")

            @pl.when(t < nfull)
            def _():
                in_copies(t, b, "wait")
                transpose(b)
                out_copy(t, b, "start")

                @pl.when(t + 2 * _NW < nfull)
                def _():
                    in_copies(t + 2 * _NW, b, "start")

        return carry

    ngrp = nfull // _NW // 2 + 2
    lax.fori_loop(0, ngrp, it_body, 0)

    vrem = v % _TV
    if vrem:
        @pl.when(wid == 0)
        def _():
            pltpu.sync_copy(tail_hbm, touts[0].at[pl.ds(0, vrem // 2)])
            pltpu.sync_copy(
                touts[0].at[pl.ds(0, vrem // 2)],
                mid_hbm.at[pl.ds(nfull * (_TV // 2), vrem // 2)])


def _emb_body(nseq, d, scale, idx_hbm, table_hbm, out_hbm, idx_v, in_bufs,
              out_bufs, sidx_bufs, gsems, osems):
    wid = _wid()
    nfb = d // 8
    nq = d // _L
    biota, cvecs = _diag_vecs()

    def stage(st, carry):
        pltpu.sync_copy(idx_hbm.at[st, wid], idx_v.at[st])
        return carry

    lax.fori_loop(0, nseq // 8, stage, 0)

    def start_gather(j, b):
        pltpu.make_async_copy(
            table_hbm.at[idx_v.at[j // 8, j % 8]], in_bufs[b],
            gsems[b]).start()

    def wait_gather(j, b):
        pltpu.make_async_copy(
            table_hbm.at[idx_v.at[j // 8, j % 8]], in_bufs[b],
            gsems[b]).wait()

    def out_copies(j, b, fn):
        for fb in range(nfb):
            getattr(pltpu.make_async_copy(
                out_bufs[b].at[pl.ds(fb * 8, 8)],
                out_hbm.at[j, fb, wid], osems[b]), fn)()

    def transpose_scale(j, b):
        src = in_bufs[b]
        dst = out_bufs[b]

        @plsc.parallel_loop(0, (_K // _L) * nq)
        def block_body(t):
            rb = t // nq
            q = t % nq
            rvec = biota + rb * _L
            for dg in range(_L):
                cols = cvecs[dg] + q * _L
                vals = plsc.load_gather(src, [rvec, cols])
                plsc.store_scatter(dst, [cols, rvec], vals * scale)

    ngroup = nseq // _NB

    del sidx_bufs
    for b in range(_NB):
        start_gather(b, b)

    def group(g, carry):
        for b in range(_NB):
            j = g * _NB + b
            wait_gather(j, b)

            @pl.when(g > 0)
            def _():
                out_copies(j - _NB, b, "wait")

            transpose_scale(j, b)
            out_copies(j, b, "start")

            @pl.when(j + _NB < nseq)
            def _():
                start_gather(j + _NB, b)

        return carry

    lax.fori_loop(0, ngroup, group, 0)

    for b in range(_NB):
        out_copies(nseq - _NB + b, b, "wait")


def kernel(input, table):
    bt, s = input.shape
    v, d = table.shape
    assert d % _L == 0 and d % 8 == 0 and v % 2 == 0
    nbb = bt // _K
    assert nbb == _NW and s % 8 == 0 and s % _NB == 0
    scale = np.float32(np.sqrt(d))

    idx4 = (input.astype(jnp.int32)
            .reshape(nbb, _K, s // 8, 8).transpose(2, 0, 3, 1))
    tt = table.T
    vrem = v % _TV
    ntail = max(vrem // 2, 1)
    tail2 = table[v - max(vrem, 2):].reshape(ntail, 2 * d)

    mesh = plsc.VectorSubcoreMesh(core_axis_name="c", subcore_axis_name="s")
    cparams = pltpu.CompilerParams(
        use_tc_tiling_on_sc=True, needs_layout_passes=False)

    relayout = pl.kernel(
        functools.partial(_tr_body, v, d),
        mesh=mesh,
        out_type=jax.ShapeDtypeStruct((v // 2, 2 * d), jnp.float32),
        scratch_types=(
            [pltpu.VMEM((d, _TV), jnp.float32) for _ in range(2)]
            + [pltpu.VMEM((_TV // 2, 2 * d), jnp.float32) for _ in range(2)]
            + [pltpu.SemaphoreType.DMA] * 4
        ),
        compiler_params=cparams,
    )

    def body(idx_hbm, table_hbm, out_hbm, idx_v, *rest):
        in_bufs = rest[:_NB]
        out_bufs = rest[_NB:2 * _NB]
        gsems = rest[2 * _NB:3 * _NB]
        osems = rest[3 * _NB:]
        _emb_body(s, d, scale, idx_hbm, table_hbm, out_hbm, idx_v, in_bufs,
                  out_bufs, None, gsems, osems)

    gather = pl.kernel(
        body,
        mesh=mesh,
        out_type=jax.ShapeDtypeStruct((s, d // 8, nbb, 8, _K), jnp.float32),
        scratch_types=(
            [pltpu.VMEM((s // 8, 8, _K), jnp.int32)]
            + [pltpu.VMEM((_K, d), jnp.float32) for _ in range(_NB)]
            + [pltpu.VMEM((d, _K), jnp.float32) for _ in range(_NB)]
            + [pltpu.SemaphoreType.DMA] * (2 * _NB)
        ),
        compiler_params=pltpu.CompilerParams(
            use_tc_tiling_on_sc=False, needs_layout_passes=False),
    )

    mid = relayout(tt, tail2)
    out5 = gather(idx4, mid.reshape(v, d))
    return out5.transpose(2, 4, 0, 1, 3).reshape(bt, s, d)


# speedup vs baseline: 1.8391x; 1.8391x over previous
import functools

import jax
import jax.numpy as jnp
import numpy as np
from jax import lax
from jax.experimental import pallas as pl
from jax.experimental.pallas import tpu as pltpu
from jax.experimental.pallas import tpu_sc as plsc

_INFO = plsc.get_sparse_core_info()
_NC = _INFO.num_cores
_NS = _INFO.num_subcores
_L = _INFO.num_lanes
_NW = _NC * _NS

_K = 128
_NB = 4
_TV = 128


def _wid():
    return lax.axis_index("s") * _NC + lax.axis_index("c")


def _diag_vecs():
    biota = lax.iota(jnp.int32, _L)
    cvecs = [(dg + biota) % _L for dg in range(_L)]
    return biota, cvecs


def _tr_body(v, d, tt_hbm, tail_hbm, mid_hbm, tin0, tin1, tout0, tout1,
             isem0, isem1, osem0, osem1):
    wid = _wid()
    tins, touts = (tin0, tin1), (tout0, tout1)
    isems, osems = (isem0, isem1), (osem0, osem1)
    nfull = v // _TV
    biota, cvecs = _diag_vecs()
    rshs = [lax.shift_right_logical(c, 1) for c in cvecs]
    cshs = [lax.shift_left(c & 1, 6) + biota for c in cvecs]

    def in_copies(t, b, fn):
        for fb in range(d // 8):
            getattr(pltpu.make_async_copy(
                tt_hbm.at[pl.ds(fb * 8, 8), pl.ds(t * _TV, _TV)],
                tins[b].at[pl.ds(fb * 8, 8)], isems[b]), fn)()

    def out_copy(t, b, fn):
        getattr(pltpu.make_async_copy(
            touts[b], mid_hbm.at[pl.ds(t * (_TV // 2), _TV // 2)],
            osems[b]), fn)()

    def transpose(b):
        src, dst = tins[b], touts[b]

        @plsc.parallel_loop(0, (d // _L) * (_TV // _L))
        def blk(u):
            rb = u // (_TV // _L)
            qb = u % (_TV // _L)
            rvec = biota + rb * _L
            for dg in range(_L):
                vals = plsc.load_gather(src, [rvec, cvecs[dg] + qb * _L])
                plsc.store_scatter(
                    dst, [rshs[dg] + qb * 8, cshs[dg] + rb * _L], vals)

    for i in range(2):
        in_copies(i * _NW + wid, i, "start")

    def it_body(g, carry):
        for b in range(2):
            i = g * 2 + b
            t = i * _NW + wid

            @pl.when((i >= 2) & (t - 2 * _NW < nfull))
            def _():
                out_copy(t - 2 * _NW, b, "wait")

            @pl.when(t < nfull)
            def _():
                in_copies(t, b, "wait")
                transpose(b)
                out_copy(t, b, "start")

                @pl.when(t + 2 * _NW < nfull)
                def _():
                    in_copies(t + 2 * _NW, b, "start")

        return carry

    ngrp = nfull // _NW // 2 + 2
    lax.fori_loop(0, ngrp, it_body, 0)

    vrem = v % _TV
    if vrem:
        @pl.when(wid == 0)
        def _():
            pltpu.sync_copy(tail_hbm, touts[0].at[pl.ds(0, vrem // 2)])
            pltpu.sync_copy(
                touts[0].at[pl.ds(0, vrem // 2)],
                mid_hbm.at[pl.ds(nfull * (_TV // 2), vrem // 2)])


def _emb_body(nseq, d, scale, idx_hbm, table_hbm, out_hbm, idx_v, in_bufs,
              out_bufs, sidx_bufs, gsems, osems):
    wid = _wid()
    nfb = d // 8
    nq = d // _L
    biota, cvecs = _diag_vecs()
    dvecs = [c * _K + biota for c in cvecs]

    def stage(st, carry):
        pltpu.sync_copy(idx_hbm.at[st, wid], idx_v.at[st])
        return carry

    lax.fori_loop(0, nseq // 8, stage, 0)

    def start_gather(j, b):
        pltpu.make_async_copy(
            table_hbm.at[idx_v.at[j // 8, j % 8]], in_bufs[b],
            gsems[b]).start()

    def wait_gather(j, b):
        pltpu.make_async_copy(
            table_hbm.at[idx_v.at[j // 8, j % 8]], in_bufs[b],
            gsems[b]).wait()

    def out_copies(j, b, fn):
        for fb in range(nfb):
            getattr(pltpu.make_async_copy(
                out_bufs[b].at[pl.ds(fb * 8 * _K, 8 * _K)],
                out_hbm.at[j, fb, wid], osems[b]), fn)()

    def transpose_scale(j, b):
        src = in_bufs[b]
        dst = out_bufs[b]

        @plsc.parallel_loop(0, (_K // _L) * nq)
        def block_body(t):
            rb = t // nq
            q = t % nq
            rvec = biota + rb * _L
            dbase = q * _L * _K + rb * _L
            for dg in range(_L):
                vals = plsc.load_gather(src, [rvec, cvecs[dg] + q * _L])
                plsc.store_scatter(dst, [dvecs[dg] + dbase], vals * scale)

    ngroup = nseq // _NB

    del sidx_bufs
    for b in range(_NB):
        start_gather(b, b)

    def group(g, carry):
        for b in range(_NB):
            j = g * _NB + b
            wait_gather(j, b)

            @pl.when(g > 0)
            def _():
                out_copies(j - _NB, b, "wait")

            transpose_scale(j, b)
            out_copies(j, b, "start")

            @pl.when(j + _NB < nseq)
            def _():
                start_gather(j + _NB, b)

        return carry

    lax.fori_loop(0, ngroup, group, 0)

    for b in range(_NB):
        out_copies(nseq - _NB + b, b, "wait")


def kernel(input, table):
    bt, s = input.shape
    v, d = table.shape
    assert d % _L == 0 and d % 8 == 0 and v % 2 == 0
    nbb = bt // _K
    assert nbb == _NW and s % 8 == 0 and s % _NB == 0
    scale = np.float32(np.sqrt(d))

    idx4 = (input.astype(jnp.int32)
            .reshape(nbb, _K, s // 8, 8).transpose(2, 0, 3, 1))
    tt = table.T
    vrem = v % _TV
    ntail = max(vrem // 2, 1)
    tail2 = table[v - max(vrem, 2):].reshape(ntail, 2 * d)

    mesh = plsc.VectorSubcoreMesh(core_axis_name="c", subcore_axis_name="s")
    cparams = pltpu.CompilerParams(
        use_tc_tiling_on_sc=True, needs_layout_passes=False)

    relayout = pl.kernel(
        functools.partial(_tr_body, v, d),
        mesh=mesh,
        out_type=jax.ShapeDtypeStruct((v // 2, 2 * d), jnp.float32),
        scratch_types=(
            [pltpu.VMEM((d, _TV), jnp.float32) for _ in range(2)]
            + [pltpu.VMEM((_TV // 2, 2 * d), jnp.float32) for _ in range(2)]
            + [pltpu.SemaphoreType.DMA] * 4
        ),
        compiler_params=cparams,
    )

    def body(idx_hbm, table_hbm, out_hbm, idx_v, *rest):
        in_bufs = rest[:_NB]
        out_bufs = rest[_NB:2 * _NB]
        gsems = rest[2 * _NB:3 * _NB]
        osems = rest[3 * _NB:]
        _emb_body(s, d, scale, idx_hbm, table_hbm, out_hbm, idx_v, in_bufs,
                  out_bufs, None, gsems, osems)

    gather = pl.kernel(
        body,
        mesh=mesh,
        out_type=jax.ShapeDtypeStruct((s, d // 8, nbb, 8 * _K), jnp.float32),
        scratch_types=(
            [pltpu.VMEM((s // 8, 8, _K), jnp.int32)]
            + [pltpu.VMEM((_K, d), jnp.float32) for _ in range(_NB)]
            + [pltpu.VMEM((d * _K,), jnp.float32) for _ in range(_NB)]
            + [pltpu.SemaphoreType.DMA] * (2 * _NB)
        ),
        compiler_params=pltpu.CompilerParams(
            use_tc_tiling_on_sc=False, needs_layout_passes=False),
    )

    mid = relayout(tt, tail2)
    out5 = gather(idx4, mid.reshape(v, d)).reshape(s, d // 8, nbb, 8, _K)
    return out5.transpose(2, 4, 0, 1, 3).reshape(bt, s, d)


# speedup vs baseline: 1.9389x; 1.0543x over previous
import functools

import jax
import jax.numpy as jnp
import numpy as np
from jax import lax
from jax.experimental import pallas as pl
from jax.experimental.pallas import tpu as pltpu
from jax.experimental.pallas import tpu_sc as plsc

_INFO = plsc.get_sparse_core_info()
_NC = _INFO.num_cores
_NS = _INFO.num_subcores
_L = _INFO.num_lanes
_NW = _NC * _NS

_K = 128
_NB = 4
_TV = 256


def _wid():
    return lax.axis_index("s") * _NC + lax.axis_index("c")


def _diag_vecs():
    biota = lax.iota(jnp.int32, _L)
    cvecs = [(dg + biota) % _L for dg in range(_L)]
    return biota, cvecs


def _tr_body(v, d, tt_hbm, tail_hbm, mid_hbm, tin0, tin1, tout0, tout1,
             isem0, isem1, osem0, osem1):
    wid = _wid()
    tins, touts = (tin0, tin1), (tout0, tout1)
    isems, osems = (isem0, isem1), (osem0, osem1)
    nfull = v // _TV
    biota, cvecs = _diag_vecs()
    rshs = [lax.shift_right_logical(c, 1) for c in cvecs]
    cshs = [lax.shift_left(c & 1, 6) + biota for c in cvecs]

    def in_copies(t, b, fn):
        for fb in range(d // 8):
            getattr(pltpu.make_async_copy(
                tt_hbm.at[pl.ds(fb * 8, 8), pl.ds(t * _TV, _TV)],
                tins[b].at[pl.ds(fb * 8, 8)], isems[b]), fn)()

    def out_copy(t, b, fn):
        getattr(pltpu.make_async_copy(
            touts[b], mid_hbm.at[pl.ds(t * (_TV // 2), _TV // 2)],
            osems[b]), fn)()

    def transpose(b):
        src, dst = tins[b], touts[b]

        @plsc.parallel_loop(0, (d // _L) * (_TV // _L))
        def blk(u):
            rb = u // (_TV // _L)
            qb = u % (_TV // _L)
            rvec = biota + rb * _L
            for dg in range(_L):
                vals = plsc.load_gather(src, [rvec, cvecs[dg] + qb * _L])
                plsc.store_scatter(
                    dst, [rshs[dg] + qb * 8, cshs[dg] + rb * _L], vals)

    for i in range(2):
        in_copies(i * _NW + wid, i, "start")

    def it_body(g, carry):
        for b in range(2):
            i = g * 2 + b
            t = i * _NW + wid

            @pl.when((i >= 2) & (t - 2 * _NW < nfull))
            def _():
                out_copy(t - 2 * _NW, b, "wait")

            @pl.when(t < nfull)
            def _():
                in_copies(t, b, "wait")
                transpose(b)
                out_copy(t, b, "start")

                @pl.when(t + 2 * _NW < nfull)
                def _():
                    in_copies(t + 2 * _NW, b, "start")

        return carry

    ngrp = nfull // _NW // 2 + 2
    lax.fori_loop(0, ngrp, it_body, 0)

    vrem = v % _TV
    if vrem:
        @pl.when(wid == 0)
        def _():
            pltpu.sync_copy(tail_hbm, touts[0].at[pl.ds(0, vrem // 2)])
            pltpu.sync_copy(
                touts[0].at[pl.ds(0, vrem // 2)],
                mid_hbm.at[pl.ds(nfull * (_TV // 2), vrem // 2)])


def _emb_body(nseq, d, scale, idx_hbm, table_hbm, out_hbm, idx_v, in_bufs,
              out_bufs, sidx_bufs, gsems, osems):
    wid = _wid()
    nfb = d // 8
    nq = d // _L
    biota, cvecs = _diag_vecs()
    dvecs = [c * _K + biota for c in cvecs]

    def stage(st, carry):
        pltpu.sync_copy(idx_hbm.at[st, wid], idx_v.at[st])
        return carry

    lax.fori_loop(0, nseq // 8, stage, 0)

    def start_gather(j, b):
        pltpu.make_async_copy(
            table_hbm.at[idx_v.at[j // 8, j % 8]], in_bufs[b],
            gsems[b]).start()

    def wait_gather(j, b):
        pltpu.make_async_copy(
            table_hbm.at[idx_v.at[j // 8, j % 8]], in_bufs[b],
            gsems[b]).wait()

    def out_copies(j, b, fn):
        for fb in range(nfb):
            getattr(pltpu.make_async_copy(
                out_bufs[b].at[pl.ds(fb * 8 * _K, 8 * _K)],
                out_hbm.at[j, fb, wid], osems[b]), fn)()

    def transpose_scale(j, b):
        src = in_bufs[b]
        dst = out_bufs[b]

        @plsc.parallel_loop(0, (_K // _L) * nq)
        def block_body(t):
            rb = t // nq
            q = t % nq
            rvec = biota + rb * _L
            dbase = q * _L * _K + rb * _L
            for dg in range(_L):
                vals = plsc.load_gather(src, [rvec, cvecs[dg] + q * _L])
                plsc.store_scatter(dst, [dvecs[dg] + dbase], vals * scale)

    ngroup = nseq // _NB

    del sidx_bufs
    for b in range(_NB):
        start_gather(b, b)

    def group(g, carry):
        for b in range(_NB):
            j = g * _NB + b
            wait_gather(j, b)

            @pl.when(g > 0)
            def _():
                out_copies(j - _NB, b, "wait")

            transpose_scale(j, b)
            out_copies(j, b, "start")

            @pl.when(j + _NB < nseq)
            def _():
                start_gather(j + _NB, b)

        return carry

    lax.fori_loop(0, ngroup, group, 0)

    for b in range(_NB):
        out_copies(nseq - _NB + b, b, "wait")


def kernel(input, table):
    bt, s = input.shape
    v, d = table.shape
    assert d % _L == 0 and d % 8 == 0 and v % 2 == 0
    nbb = bt // _K
    assert nbb == _NW and s % 8 == 0 and s % _NB == 0
    scale = np.float32(np.sqrt(d))

    idx4 = (input.astype(jnp.int32)
            .reshape(nbb, _K, s // 8, 8).transpose(2, 0, 3, 1))
    tt = table.T
    vrem = v % _TV
    ntail = max(vrem // 2, 1)
    tail2 = table[v - max(vrem, 2):].reshape(ntail, 2 * d)

    mesh = plsc.VectorSubcoreMesh(core_axis_name="c", subcore_axis_name="s")
    cparams = pltpu.CompilerParams(
        use_tc_tiling_on_sc=True, needs_layout_passes=False)

    relayout = pl.kernel(
        functools.partial(_tr_body, v, d),
        mesh=mesh,
        out_type=jax.ShapeDtypeStruct((v // 2, 2 * d), jnp.float32),
        scratch_types=(
            [pltpu.VMEM((d, _TV), jnp.float32) for _ in range(2)]
            + [pltpu.VMEM((_TV // 2, 2 * d), jnp.float32) for _ in range(2)]
            + [pltpu.SemaphoreType.DMA] * 4
        ),
        compiler_params=cparams,
    )

    def body(idx_hbm, table_hbm, out_hbm, idx_v, *rest):
        in_bufs = rest[:_NB]
        out_bufs = rest[_NB:2 * _NB]
        gsems = rest[2 * _NB:3 * _NB]
        osems = rest[3 * _NB:]
        _emb_body(s, d, scale, idx_hbm, table_hbm, out_hbm, idx_v, in_bufs,
                  out_bufs, None, gsems, osems)

    gather = pl.kernel(
        body,
        mesh=mesh,
        out_type=jax.ShapeDtypeStruct((s, d // 8, nbb, 8 * _K), jnp.float32),
        scratch_types=(
            [pltpu.VMEM((s // 8, 8, _K), jnp.int32)]
            + [pltpu.VMEM((_K, d), jnp.float32) for _ in range(_NB)]
            + [pltpu.VMEM((d * _K,), jnp.float32) for _ in range(_NB)]
            + [pltpu.SemaphoreType.DMA] * (2 * _NB)
        ),
        compiler_params=pltpu.CompilerParams(
            use_tc_tiling_on_sc=False, needs_layout_passes=False),
    )

    mid = relayout(tt, tail2)
    out5 = gather(idx4, mid.reshape(v, d)).reshape(s, d // 8, nbb, 8, _K)
    return out5.transpose(2, 4, 0, 1, 3).reshape(bt, s, d)


# speedup vs baseline: 2.0344x; 1.0493x over previous
import functools

import jax
import jax.numpy as jnp
import numpy as np
from jax import lax
from jax.experimental import pallas as pl
from jax.experimental.pallas import tpu as pltpu
from jax.experimental.pallas import tpu_sc as plsc

_INFO = plsc.get_sparse_core_info()
_NC = _INFO.num_cores
_NS = _INFO.num_subcores
_L = _INFO.num_lanes
_NW = _NC * _NS

_K = 128
_NB = 4
_TV = 384


def _wid():
    return lax.axis_index("s") * _NC + lax.axis_index("c")


def _diag_vecs():
    biota = lax.iota(jnp.int32, _L)
    cvecs = [(dg + biota) % _L for dg in range(_L)]
    return biota, cvecs


def _tr_body(v, d, tt_hbm, tail_hbm, mid_hbm, tin0, tin1, tout0, tout1,
             isem0, isem1, osem0, osem1):
    wid = _wid()
    tins, touts = (tin0, tin1), (tout0, tout1)
    isems, osems = (isem0, isem1), (osem0, osem1)
    nfull = v // _TV
    biota, cvecs = _diag_vecs()
    rshs = [lax.shift_right_logical(c, 1) for c in cvecs]
    cshs = [lax.shift_left(c & 1, 6) + biota for c in cvecs]

    def in_copies(t, b, fn):
        for fb in range(d // 8):
            getattr(pltpu.make_async_copy(
                tt_hbm.at[pl.ds(fb * 8, 8), pl.ds(t * _TV, _TV)],
                tins[b].at[pl.ds(fb * 8, 8)], isems[b]), fn)()

    def out_copy(t, b, fn):
        getattr(pltpu.make_async_copy(
            touts[b], mid_hbm.at[pl.ds(t * (_TV // 2), _TV // 2)],
            osems[b]), fn)()

    def transpose(b):
        src, dst = tins[b], touts[b]

        @plsc.parallel_loop(0, (d // _L) * (_TV // _L))
        def blk(u):
            rb = u // (_TV // _L)
            qb = u % (_TV // _L)
            rvec = biota + rb * _L
            for dg in range(_L):
                vals = plsc.load_gather(src, [rvec, cvecs[dg] + qb * _L])
                plsc.store_scatter(
                    dst, [rshs[dg] + qb * 8, cshs[dg] + rb * _L], vals)

    for i in range(2):
        in_copies(i * _NW + wid, i, "start")

    def it_body(g, carry):
        for b in range(2):
            i = g * 2 + b
            t = i * _NW + wid

            @pl.when((i >= 2) & (t - 2 * _NW < nfull))
            def _():
                out_copy(t - 2 * _NW, b, "wait")

            @pl.when(t < nfull)
            def _():
                in_copies(t, b, "wait")
                transpose(b)
                out_copy(t, b, "start")

                @pl.when(t + 2 * _NW < nfull)
                def _():
                    in_copies(t + 2 * _NW, b, "start")

        return carry

    ngrp = nfull // _NW // 2 + 2
    lax.fori_loop(0, ngrp, it_body, 0)

    vrem = v % _TV
    if vrem:
        @pl.when(wid == 0)
        def _():
            pltpu.sync_copy(tail_hbm, touts[0].at[pl.ds(0, vrem // 2)])
            pltpu.sync_copy(
                touts[0].at[pl.ds(0, vrem // 2)],
                mid_hbm.at[pl.ds(nfull * (_TV // 2), vrem // 2)])


def _emb_body(nseq, d, scale, idx_hbm, table_hbm, out_hbm, idx_v, in_bufs,
              out_bufs, sidx_bufs, gsems, osems):
    wid = _wid()
    nfb = d // 8
    nq = d // _L
    biota, cvecs = _diag_vecs()
    dvecs = [c * _K + biota for c in cvecs]

    def stage(st, carry):
        pltpu.sync_copy(idx_hbm.at[st, wid], idx_v.at[st])
        return carry

    lax.fori_loop(0, nseq // 8, stage, 0)

    def start_gather(j, b):
        pltpu.make_async_copy(
            table_hbm.at[idx_v.at[j // 8, j % 8]], in_bufs[b],
            gsems[b]).start()

    def wait_gather(j, b):
        pltpu.make_async_copy(
            table_hbm.at[idx_v.at[j // 8, j % 8]], in_bufs[b],
            gsems[b]).wait()

    def out_copies(j, b, fn):
        for fb in range(nfb):
            getattr(pltpu.make_async_copy(
                out_bufs[b].at[pl.ds(fb * 8 * _K, 8 * _K)],
                out_hbm.at[j, fb, wid], osems[b]), fn)()

    def transpose_scale(j, b):
        src = in_bufs[b]
        dst = out_bufs[b]

        @plsc.parallel_loop(0, (_K // _L) * nq)
        def block_body(t):
            rb = t // nq
            q = t % nq
            rvec = biota + rb * _L
            dbase = q * _L * _K + rb * _L
            for dg in range(_L):
                vals = plsc.load_gather(src, [rvec, cvecs[dg] + q * _L])
                plsc.store_scatter(dst, [dvecs[dg] + dbase], vals * scale)

    ngroup = nseq // _NB

    del sidx_bufs
    for b in range(_NB):
        start_gather(b, b)

    def group(g, carry):
        for b in range(_NB):
            j = g * _NB + b
            wait_gather(j, b)

            @pl.when(g > 0)
            def _():
                out_copies(j - _NB, b, "wait")

            transpose_scale(j, b)
            out_copies(j, b, "start")

            @pl.when(j + _NB < nseq)
            def _():
                start_gather(j + _NB, b)

        return carry

    lax.fori_loop(0, ngroup, group, 0)

    for b in range(_NB):
        out_copies(nseq - _NB + b, b, "wait")


def kernel(input, table):
    bt, s = input.shape
    v, d = table.shape
    assert d % _L == 0 and d % 8 == 0 and v % 2 == 0
    nbb = bt // _K
    assert nbb == _NW and s % 8 == 0 and s % _NB == 0
    scale = np.float32(np.sqrt(d))

    idx4 = (input.astype(jnp.int32)
            .reshape(nbb, _K, s // 8, 8).transpose(2, 0, 3, 1))
    tt = table.T
    vrem = v % _TV
    ntail = max(vrem // 2, 1)
    tail2 = table[v - max(vrem, 2):].reshape(ntail, 2 * d)

    mesh = plsc.VectorSubcoreMesh(core_axis_name="c", subcore_axis_name="s")
    cparams = pltpu.CompilerParams(
        use_tc_tiling_on_sc=True, needs_layout_passes=False)

    relayout = pl.kernel(
        functools.partial(_tr_body, v, d),
        mesh=mesh,
        out_type=jax.ShapeDtypeStruct((v // 2, 2 * d), jnp.float32),
        scratch_types=(
            [pltpu.VMEM((d, _TV), jnp.float32) for _ in range(2)]
            + [pltpu.VMEM((_TV // 2, 2 * d), jnp.float32) for _ in range(2)]
            + [pltpu.SemaphoreType.DMA] * 4
        ),
        compiler_params=cparams,
    )

    def body(idx_hbm, table_hbm, out_hbm, idx_v, *rest):
        in_bufs = rest[:_NB]
        out_bufs = rest[_NB:2 * _NB]
        gsems = rest[2 * _NB:3 * _NB]
        osems = rest[3 * _NB:]
        _emb_body(s, d, scale, idx_hbm, table_hbm, out_hbm, idx_v, in_bufs,
                  out_bufs, None, gsems, osems)

    gather = pl.kernel(
        body,
        mesh=mesh,
        out_type=jax.ShapeDtypeStruct((s, d // 8, nbb, 8 * _K), jnp.float32),
        scratch_types=(
            [pltpu.VMEM((s // 8, 8, _K), jnp.int32)]
            + [pltpu.VMEM((_K, d), jnp.float32) for _ in range(_NB)]
            + [pltpu.VMEM((d * _K,), jnp.float32) for _ in range(_NB)]
            + [pltpu.SemaphoreType.DMA] * (2 * _NB)
        ),
        compiler_params=pltpu.CompilerParams(
            use_tc_tiling_on_sc=False, needs_layout_passes=False),
    )

    mid = relayout(tt, tail2)
    out5 = gather(idx4, mid.reshape(v, d)).reshape(s, d // 8, nbb, 8, _K)
    return out5.transpose(2, 4, 0, 1, 3).reshape(bt, s, d)
